# baseline (speedup 1.0000x reference)
import jax
import jax.numpy as jnp
from jax.experimental import pallas as pl
from jax.experimental.pallas import tpu as pltpu


OUTNUM = 14
GROUP = 4
C_IN = 2048
C_MID = 256
FC_IN = 1024
FC_PAD = 128
HW = 49

IMG = 64
TB = 32
M_ROWS = TB * IMG
PAD = 16
FC_ROWS = TB // GROUP


def _fused_kernel(x_ref, w_ref, scale_ref, shift_ref, mask_ref, fcw_ref,
                  fcb_ref, o_ref, xr_ref, acc_ref):
    @pl.when(pl.program_id(0) == 0)
    def _init():
        xr_ref[...] = jnp.zeros_like(xr_ref)

    for m in range(TB):
        for i in range(7):
            xr_ref[m * IMG + 8 * i:m * IMG + 8 * i + 7, :] = \
                x_ref[m, 7 * i:7 * i + 7, :].astype(jnp.bfloat16)

    acc_ref[...] = jnp.zeros_like(acc_ref)
    for di in range(3):
        for dj in range(3):
            off = (di - 1) * 8 + (dj - 1)
            acc_ref[PAD - off:PAD - off + M_ROWS, :] += jnp.dot(
                xr_ref[...], w_ref[di * 3 + dj],
                preferred_element_type=jnp.float32)

    y = jnp.maximum(acc_ref[PAD:PAD + M_ROWS, :] * scale_ref[...]
                    + shift_ref[...], 0.0)
    y = y * mask_ref[...]
    pooled = [jnp.max(y[m * IMG:(m + 1) * IMG, :], axis=0, keepdims=True)
              for m in range(TB)]
    rows = [jnp.concatenate(pooled[g * GROUP:(g + 1) * GROUP], axis=1)
            for g in range(FC_ROWS)]
    feats = jnp.concatenate(rows, axis=0).astype(jnp.bfloat16)
    o_ref[0] = (jnp.dot(feats, fcw_ref[...],
                        preferred_element_type=jnp.float32) + fcb_ref[...])


def kernel(x_nchw, conv_w9, conv_scale, conv_shift, valid_mask, fc_w, fc_b):
    N, C, H, W = x_nchw.shape
    assert C == C_IN and H == 7 and W == 7 and N % TB == 0
    nblk = N // TB
    G = N // GROUP

    xt = jnp.transpose(x_nchw.reshape(N, C_IN, HW), (0, 2, 1))
    t = jnp.arange(M_ROWS) % IMG
    mask = (((t % 8) < 7) & (t < 56)).astype(jnp.float32).reshape(M_ROWS, 1)

    out = pl.pallas_call(
        _fused_kernel,
        out_shape=jax.ShapeDtypeStruct((nblk, FC_ROWS, FC_PAD), jnp.float32),
        grid=(nblk,),
        in_specs=[
            pl.BlockSpec((TB, HW, C_IN), lambda i: (i, 0, 0)),
            pl.BlockSpec((9, C_IN, C_MID), lambda i: (0, 0, 0)),
            pl.BlockSpec((1, C_MID), lambda i: (0, 0)),
            pl.BlockSpec((1, C_MID), lambda i: (0, 0)),
            pl.BlockSpec((M_ROWS, 1), lambda i: (0, 0)),
            pl.BlockSpec((FC_IN, FC_PAD), lambda i: (0, 0)),
            pl.BlockSpec((1, FC_PAD), lambda i: (0, 0)),
        ],
        out_specs=pl.BlockSpec((1, FC_ROWS, FC_PAD), lambda i: (i, 0, 0)),
        scratch_shapes=[
            pltpu.VMEM((M_ROWS, C_IN), jnp.bfloat16),
            pltpu.VMEM((M_ROWS + 2 * PAD, C_MID), jnp.float32),
        ],
        compiler_params=pltpu.CompilerParams(
            dimension_semantics=("parallel",),
            vmem_limit_bytes=100 * 1024 * 1024),
    )(xt, conv_w9, conv_scale, conv_shift, mask, fc_w, fc_b)

    return out.reshape(G, FC_PAD)[:, :OUTNUM]


# speedup vs baseline: 1.0309x; 1.0309x over previous
import jax
import jax.numpy as jnp
from jax.experimental import pallas as pl
from jax.experimental.pallas import tpu as pltpu


OUTNUM = 14
GROUP = 4
C_IN = 2048
C_MID = 256
FC_IN = 1024
FC_PAD = 128
HW = 49

IMG = 64
TB = 16
M_ROWS = TB * IMG
PAD = 16
FC_ROWS = TB // GROUP


def _fused_kernel(x_ref, w_ref, scale_ref, shift_ref, mask_ref, fcw_ref,
                  fcb_ref, o_ref, xr_ref, acc_ref):
    @pl.when(pl.program_id(0) == 0)
    def _init():
        xr_ref[...] = jnp.zeros_like(xr_ref)

    for m in range(TB):
        for i in range(7):
            xr_ref[m * IMG + 8 * i:m * IMG + 8 * i + 7, :] = \
                x_ref[m, 7 * i:7 * i + 7, :].astype(jnp.bfloat16)

    acc_ref[...] = jnp.zeros_like(acc_ref)
    for di in range(3):
        for dj in range(3):
            off = (di - 1) * 8 + (dj - 1)
            acc_ref[PAD - off:PAD - off + M_ROWS, :] += jnp.dot(
                xr_ref[...], w_ref[di * 3 + dj],
                preferred_element_type=jnp.float32)

    y = jnp.maximum(acc_ref[PAD:PAD + M_ROWS, :] * scale_ref[...]
                    + shift_ref[...], 0.0)
    y = y * mask_ref[...]
    pooled = [jnp.max(y[m * IMG:(m + 1) * IMG, :], axis=0, keepdims=True)
              for m in range(TB)]
    rows = [jnp.concatenate(pooled[g * GROUP:(g + 1) * GROUP], axis=1)
            for g in range(FC_ROWS)]
    feats = jnp.concatenate(rows, axis=0).astype(jnp.bfloat16)
    o_ref[0] = (jnp.dot(feats, fcw_ref[...],
                        preferred_element_type=jnp.float32) + fcb_ref[...])


def kernel(x_nchw, conv_w9, conv_scale, conv_shift, valid_mask, fc_w, fc_b):
    N, C, H, W = x_nchw.shape
    assert C == C_IN and H == 7 and W == 7 and N % TB == 0
    nblk = N // TB
    G = N // GROUP

    xt = jnp.transpose(x_nchw.reshape(N, C_IN, HW), (0, 2, 1))
    t = jnp.arange(M_ROWS) % IMG
    mask = (((t % 8) < 7) & (t < 56)).astype(jnp.float32).reshape(M_ROWS, 1)

    out = pl.pallas_call(
        _fused_kernel,
        out_shape=jax.ShapeDtypeStruct((nblk, FC_ROWS, FC_PAD), jnp.float32),
        grid=(nblk,),
        in_specs=[
            pl.BlockSpec((TB, HW, C_IN), lambda i: (i, 0, 0)),
            pl.BlockSpec((9, C_IN, C_MID), lambda i: (0, 0, 0)),
            pl.BlockSpec((1, C_MID), lambda i: (0, 0)),
            pl.BlockSpec((1, C_MID), lambda i: (0, 0)),
            pl.BlockSpec((M_ROWS, 1), lambda i: (0, 0)),
            pl.BlockSpec((FC_IN, FC_PAD), lambda i: (0, 0)),
            pl.BlockSpec((1, FC_PAD), lambda i: (0, 0)),
        ],
        out_specs=pl.BlockSpec((1, FC_ROWS, FC_PAD), lambda i: (i, 0, 0)),
        scratch_shapes=[
            pltpu.VMEM((M_ROWS, C_IN), jnp.bfloat16),
            pltpu.VMEM((M_ROWS + 2 * PAD, C_MID), jnp.float32),
        ],
        compiler_params=pltpu.CompilerParams(
            dimension_semantics=("parallel",),
            vmem_limit_bytes=100 * 1024 * 1024),
    )(xt, conv_w9, conv_scale, conv_shift, mask, fc_w, fc_b)

    return out.reshape(G, FC_PAD)[:, :OUTNUM]
